# baseline (device time: 12599 ns/iter reference)
import jax
import jax.numpy as jnp
from jax import lax
from jax.experimental import pallas as pl
from jax.experimental.pallas import tpu as pltpu

Y_SIZE = 2
NCH = 8


def kernel(x):
    m, n = x.shape
    n_global = n * Y_SIZE
    mc = m // NCH
    mr = m // 128
    cr = mc // 128

    def body(
        x_hbm,
        out_ref,
        buf,
        copy_sems,
        send_buf,
        recv_buf,
        send_sems,
        recv_sems,
    ):
        my_x = lax.axis_index("x")
        my_y = lax.axis_index("y")
        nbr = (my_x, 1 - my_y)

        def chunk_copy(k):
            return pltpu.make_async_copy(
                x_hbm.at[pl.ds(k * mc, mc), :],
                buf.at[k % 2],
                copy_sems.at[k % 2],
            )

        chunk_copy(0).start()

        barrier_sem = pltpu.get_barrier_semaphore()
        pl.semaphore_signal(
            barrier_sem,
            inc=1,
            device_id=nbr,
            device_id_type=pl.DeviceIdType.MESH,
        )
        pl.semaphore_wait(barrier_sem, 1)

        rdmas = []
        for k in range(NCH):
            if k + 1 < NCH:
                chunk_copy(k + 1).start()
            chunk_copy(k).wait()

            blk = buf[k % 2]
            s = blk[:, 0:128]
            for c in range(128, n, 128):
                s = s + blk[:, c : c + 128]

            for i in range(cr):
                send_buf[k * cr + i, :] = jnp.sum(
                    s[i * 128 : (i + 1) * 128, :], axis=1
                )

            rdma = pltpu.make_async_remote_copy(
                src_ref=send_buf.at[pl.ds(k * cr, cr), :],
                dst_ref=recv_buf.at[pl.ds(k * cr, cr), :],
                send_sem=send_sems.at[k],
                recv_sem=recv_sems.at[k],
                device_id=nbr,
                device_id_type=pl.DeviceIdType.MESH,
            )
            rdma.start()
            rdmas.append(rdma)

        for rdma in rdmas:
            rdma.wait_send()
            rdma.wait_recv()

        out_ref[:, :] = (send_buf[:, :] + recv_buf[:, :]) * (1.0 / n_global)

    compact = pl.pallas_call(
        body,
        out_shape=jax.ShapeDtypeStruct((mr, 128), jnp.float32),
        in_specs=[pl.BlockSpec(memory_space=pl.ANY)],
        out_specs=pl.BlockSpec(memory_space=pltpu.VMEM),
        scratch_shapes=[
            pltpu.VMEM((2, mc, n), jnp.float32),
            pltpu.SemaphoreType.DMA((2,)),
            pltpu.VMEM((mr, 128), jnp.float32),
            pltpu.VMEM((mr, 128), jnp.float32),
            pltpu.SemaphoreType.DMA((NCH,)),
            pltpu.SemaphoreType.DMA((NCH,)),
        ],
        compiler_params=pltpu.CompilerParams(collective_id=0),
    )(x)
    return compact.reshape(m, 1)


# device time: 12280 ns/iter; 1.0260x vs baseline; 1.0260x over previous
import jax
import jax.numpy as jnp
from jax import lax
from jax.experimental import pallas as pl
from jax.experimental.pallas import tpu as pltpu

Y_SIZE = 2
NCH = 8


def kernel(x):
    m, n = x.shape
    n_global = n * Y_SIZE
    mc = m // NCH
    mr = m // 128
    cr = mc // 128

    def body(
        x_hbm,
        out_ref,
        buf,
        copy_sems,
        send_buf,
        recv_buf,
        send_sems,
        recv_sems,
    ):
        my_x = lax.axis_index("x")
        my_y = lax.axis_index("y")
        nbr = (my_x, 1 - my_y)

        def chunk_copy(k):
            return pltpu.make_async_copy(
                x_hbm.at[pl.ds(k * mc, mc), :],
                buf.at[k % 2],
                copy_sems.at[k % 2],
            )

        chunk_copy(0).start()

        barrier_sem = pltpu.get_barrier_semaphore()
        pl.semaphore_signal(
            barrier_sem,
            inc=1,
            device_id=nbr,
            device_id_type=pl.DeviceIdType.MESH,
        )
        pl.semaphore_wait(barrier_sem, 1)

        rdmas = []
        for k in range(NCH):
            if k + 1 < NCH:
                chunk_copy(k + 1).start()
            chunk_copy(k).wait()

            blk = buf[k % 2]
            s = blk[:, 0:128]
            for c in range(128, n, 128):
                s = s + blk[:, c : c + 128]

            for i in range(cr):
                send_buf[k * cr + i, :] = jnp.sum(
                    s[i * 128 : (i + 1) * 128, :], axis=1
                )

            rdma = pltpu.make_async_remote_copy(
                src_ref=send_buf.at[pl.ds(k * cr, cr), :],
                dst_ref=recv_buf.at[pl.ds(k * cr, cr), :],
                send_sem=send_sems.at[k],
                recv_sem=recv_sems.at[k],
                device_id=nbr,
                device_id_type=pl.DeviceIdType.MESH,
            )
            rdma.start()
            rdmas.append(rdma)

        for rdma in rdmas:
            rdma.wait_send()
            rdma.wait_recv()

        out_ref[:, :] = (send_buf[:, :] + recv_buf[:, :]) * (1.0 / n_global)

    compact = pl.pallas_call(
        body,
        out_shape=jax.ShapeDtypeStruct((mr, 128), jnp.float32),
        in_specs=[pl.BlockSpec(memory_space=pl.ANY)],
        out_specs=pl.BlockSpec(memory_space=pltpu.VMEM),
        scratch_shapes=[
            pltpu.VMEM((2, mc, n), jnp.float32),
            pltpu.SemaphoreType.DMA((2,)),
            pltpu.VMEM((mr, 128), jnp.float32),
            pltpu.VMEM((mr, 128), jnp.float32),
            pltpu.SemaphoreType.DMA((NCH,)),
            pltpu.SemaphoreType.DMA((NCH,)),
        ],
        compiler_params=pltpu.CompilerParams(collective_id=0),
    )(pltpu.with_memory_space_constraint(x, pltpu.MemorySpace.HBM))
    return compact.reshape(m, 1)


# device time: 11439 ns/iter; 1.1014x vs baseline; 1.0735x over previous
import jax
import jax.numpy as jnp
from jax import lax
from jax.experimental import pallas as pl
from jax.experimental.pallas import tpu as pltpu

Y_SIZE = 2
NCH = 16
DEPTH = 4


def kernel(x):
    m, n = x.shape
    n_global = n * Y_SIZE
    mc = m // NCH
    mr = m // 128
    cr = mc // 128

    def body(
        x_hbm,
        out_ref,
        buf,
        copy_sems,
        send_buf,
        recv_buf,
        send_sems,
        recv_sems,
    ):
        my_x = lax.axis_index("x")
        my_y = lax.axis_index("y")
        nbr = (my_x, 1 - my_y)

        def chunk_copy(k):
            return pltpu.make_async_copy(
                x_hbm.at[pl.ds(k * mc, mc), :],
                buf.at[k % DEPTH],
                copy_sems.at[k % DEPTH],
            )

        for d in range(DEPTH):
            chunk_copy(d).start()

        barrier_sem = pltpu.get_barrier_semaphore()
        pl.semaphore_signal(
            barrier_sem,
            inc=1,
            device_id=nbr,
            device_id_type=pl.DeviceIdType.MESH,
        )
        pl.semaphore_wait(barrier_sem, 1)

        rdmas = []
        for k in range(NCH):
            chunk_copy(k).wait()

            blk = buf[k % DEPTH]
            s = blk[:, 0:128]
            for c in range(128, n, 128):
                s = s + blk[:, c : c + 128]

            for i in range(cr):
                send_buf[k * cr + i, :] = jnp.sum(
                    s[i * 128 : (i + 1) * 128, :], axis=1
                )

            rdma = pltpu.make_async_remote_copy(
                src_ref=send_buf.at[pl.ds(k * cr, cr), :],
                dst_ref=recv_buf.at[pl.ds(k * cr, cr), :],
                send_sem=send_sems.at[k],
                recv_sem=recv_sems.at[k],
                device_id=nbr,
                device_id_type=pl.DeviceIdType.MESH,
            )
            rdma.start()
            rdmas.append(rdma)

            if k + DEPTH < NCH:
                chunk_copy(k + DEPTH).start()

        for rdma in rdmas:
            rdma.wait_send()
            rdma.wait_recv()

        out_ref[:, :] = (send_buf[:, :] + recv_buf[:, :]) * (1.0 / n_global)

    compact = pl.pallas_call(
        body,
        out_shape=jax.ShapeDtypeStruct((mr, 128), jnp.float32),
        in_specs=[pl.BlockSpec(memory_space=pl.ANY)],
        out_specs=pl.BlockSpec(memory_space=pltpu.VMEM),
        scratch_shapes=[
            pltpu.VMEM((DEPTH, mc, n), jnp.float32),
            pltpu.SemaphoreType.DMA((DEPTH,)),
            pltpu.VMEM((mr, 128), jnp.float32),
            pltpu.VMEM((mr, 128), jnp.float32),
            pltpu.SemaphoreType.DMA((NCH,)),
            pltpu.SemaphoreType.DMA((NCH,)),
        ],
        compiler_params=pltpu.CompilerParams(collective_id=0),
    )(pltpu.with_memory_space_constraint(x, pltpu.MemorySpace.HBM))
    return compact.reshape(m, 1)


# device time: 10282 ns/iter; 1.2253x vs baseline; 1.1125x over previous
import jax
import jax.numpy as jnp
from jax import lax
from jax.experimental import pallas as pl
from jax.experimental.pallas import tpu as pltpu

Y_SIZE = 2
NCH = 8
DEPTH = 4


def kernel(x):
    m, n = x.shape
    n_global = n * Y_SIZE
    mc = m // NCH
    mr = m // 128
    cr = mc // 128

    def body(
        x_hbm,
        out_ref,
        buf,
        copy_sems,
        send_buf,
        recv_buf,
        send_sems,
        recv_sems,
    ):
        my_x = lax.axis_index("x")
        my_y = lax.axis_index("y")
        nbr = (my_x, 1 - my_y)

        def chunk_copy(k):
            return pltpu.make_async_copy(
                x_hbm.at[pl.ds(k * mc, mc), :],
                buf.at[k % DEPTH],
                copy_sems.at[k % DEPTH],
            )

        for d in range(DEPTH):
            chunk_copy(d).start()

        barrier_sem = pltpu.get_barrier_semaphore()
        pl.semaphore_signal(
            barrier_sem,
            inc=1,
            device_id=nbr,
            device_id_type=pl.DeviceIdType.MESH,
        )
        pl.semaphore_wait(barrier_sem, 1)

        rdmas = []
        for k in range(NCH):
            chunk_copy(k).wait()

            blk = buf[k % DEPTH]
            s = blk[:, 0:128]
            for c in range(128, n, 128):
                s = s + blk[:, c : c + 128]

            for i in range(cr):
                send_buf[k * cr + i, :] = jnp.sum(
                    s[i * 128 : (i + 1) * 128, :], axis=1
                )

            rdma = pltpu.make_async_remote_copy(
                src_ref=send_buf.at[pl.ds(k * cr, cr), :],
                dst_ref=recv_buf.at[pl.ds(k * cr, cr), :],
                send_sem=send_sems.at[k],
                recv_sem=recv_sems.at[k],
                device_id=nbr,
                device_id_type=pl.DeviceIdType.MESH,
            )
            rdma.start()
            rdmas.append(rdma)

            if k + DEPTH < NCH:
                chunk_copy(k + DEPTH).start()

        for rdma in rdmas:
            rdma.wait_send()
            rdma.wait_recv()

        out_ref[:, :] = (send_buf[:, :] + recv_buf[:, :]) * (1.0 / n_global)

    compact = pl.pallas_call(
        body,
        out_shape=jax.ShapeDtypeStruct((mr, 128), jnp.float32),
        in_specs=[pl.BlockSpec(memory_space=pl.ANY)],
        out_specs=pl.BlockSpec(memory_space=pltpu.VMEM),
        scratch_shapes=[
            pltpu.VMEM((DEPTH, mc, n), jnp.float32),
            pltpu.SemaphoreType.DMA((DEPTH,)),
            pltpu.VMEM((mr, 128), jnp.float32),
            pltpu.VMEM((mr, 128), jnp.float32),
            pltpu.SemaphoreType.DMA((NCH,)),
            pltpu.SemaphoreType.DMA((NCH,)),
        ],
        compiler_params=pltpu.CompilerParams(collective_id=0),
    )(pltpu.with_memory_space_constraint(x, pltpu.MemorySpace.HBM))
    return compact.reshape(m, 1)
